# Initial kernel scaffold; baseline (speedup 1.0000x reference)
import jax
import jax.numpy as jnp
from jax.experimental import pallas as pl


def kernel(im, im_l, s, s_l):
    raise NotImplementedError("write your pallas kernel here")


# speedup vs baseline: 1.7659x; 1.7659x over previous
import functools

import jax
import jax.numpy as jnp
from jax.experimental import pallas as pl
from jax.experimental.pallas import tpu as pltpu

LAMBDA_SOFTMAX = 9.0
LAMBDA_LSE = 6.0
MARGIN = 0.2
EPS = 1e-8

_INTERPRET = False


def _wnorm_kernel(s_ref, o_ref):
    x = s_ref[...]
    o_ref[...] = jnp.sqrt(jnp.sum(x * x, axis=1, keepdims=True))


def _scores_kernel(im_ref, capT_ref, mask_ref, w1_ref, E_ref, ET_ref, o_ref):
    imr = im_ref[0]
    mask = mask_ref[...]
    raw = jnp.dot(imr, capT_ref[...], preferred_element_type=jnp.float32)
    G = jax.lax.dot_general(imr, imr, (((1,), (1,)), ((), ())),
                            preferred_element_type=jnp.float32)
    lk = jnp.where(raw >= 0, raw, 0.1 * raw) * mask
    nsum = jnp.dot(lk * lk, E_ref[...], preferred_element_type=jnp.float32)
    ninv = 1.0 / (jnp.sqrt(nsum) + EPS)
    den = jnp.dot(ninv, ET_ref[...], preferred_element_type=jnp.float32)
    x = lk * den * LAMBDA_SOFTMAX
    m = jnp.max(x, axis=0, keepdims=True)
    e = jnp.exp(x - m)
    ssum = jnp.sum(e, axis=0, keepdims=True)
    a = e * (1.0 / ssum)
    w12 = jnp.sum(a * raw, axis=0, keepdims=True)
    v = jnp.dot(G, a, preferred_element_type=jnp.float32)
    w2 = jnp.sqrt(jnp.sum(a * v, axis=0, keepdims=True))
    sim = w12 / jnp.maximum(w1_ref[...] * w2, EPS)
    expd = jnp.exp(sim * LAMBDA_LSE) * mask
    ssc = jnp.dot(expd, E_ref[...], preferred_element_type=jnp.float32)
    o_ref[0, 0] = jnp.log(ssc) / LAMBDA_LSE


def _loss_kernel(sc_ref, o_ref):
    sc = sc_ref[...]
    B = sc.shape[0]
    ri = jax.lax.broadcasted_iota(jnp.int32, (B, B), 0)
    ci = jax.lax.broadcasted_iota(jnp.int32, (B, B), 1)
    eye = ri == ci
    diag_col = jnp.sum(jnp.where(eye, sc, 0.0), axis=1, keepdims=True)
    diag_row = jnp.sum(jnp.where(eye, sc, 0.0), axis=0, keepdims=True)
    cs = jnp.maximum(MARGIN + sc - diag_col, 0.0)
    cim = jnp.maximum(MARGIN + sc - diag_row, 0.0)
    cs = jnp.where(eye, 0.0, cs)
    cim = jnp.where(eye, 0.0, cim)
    s1 = jnp.sum(jnp.max(cs, axis=1, keepdims=True), axis=0, keepdims=True)
    s2 = jnp.sum(jnp.max(cim, axis=0, keepdims=True), axis=1, keepdims=True)
    o_ref[...] = s1 + s2


@functools.partial(jax.jit, static_argnames=())
def kernel(im, im_l, s, s_l):
    B, R, D = im.shape
    W = s.shape[1]
    NJ = 2
    CJ = B // NJ
    LJ = CJ * W

    s_flat = s.reshape(B * W, D)

    GW = 8
    w1_col = pl.pallas_call(
        _wnorm_kernel,
        grid=(GW,),
        in_specs=[pl.BlockSpec((B * W // GW, D), lambda g: (g, 0))],
        out_specs=pl.BlockSpec((B * W // GW, 1), lambda g: (g, 0)),
        out_shape=jax.ShapeDtypeStruct((B * W, 1), jnp.float32),
        name="word_norms",
        interpret=_INTERPRET,
    )(s_flat)
    w1_flat = w1_col.reshape(1, B * W)

    capT = s_flat.T
    wpos = jnp.tile(jnp.arange(W, dtype=jnp.int32), B)
    slv = jnp.repeat(s_l.astype(jnp.int32), W)
    mask_flat = (wpos < slv).astype(jnp.float32).reshape(1, B * W)
    E = (jnp.arange(LJ, dtype=jnp.int32)[:, None] // W
         == jnp.arange(CJ, dtype=jnp.int32)[None, :]).astype(jnp.float32)
    ET = E.T

    scores4 = pl.pallas_call(
        _scores_kernel,
        grid=(NJ, B),
        in_specs=[
            pl.BlockSpec((1, R, D), lambda j, i: (i, 0, 0)),
            pl.BlockSpec((D, LJ), lambda j, i: (0, j)),
            pl.BlockSpec((1, LJ), lambda j, i: (0, j)),
            pl.BlockSpec((1, LJ), lambda j, i: (0, j)),
            pl.BlockSpec((LJ, CJ), lambda j, i: (0, 0)),
            pl.BlockSpec((CJ, LJ), lambda j, i: (0, 0)),
        ],
        out_specs=pl.BlockSpec((1, 1, 1, CJ), lambda j, i: (j, i, 0, 0)),
        out_shape=jax.ShapeDtypeStruct((NJ, B, 1, CJ), jnp.float32),
        compiler_params=pltpu.CompilerParams(
            dimension_semantics=("parallel", "arbitrary"),
            vmem_limit_bytes=56 * 1024 * 1024,
        ),
        name="caption_scores",
        interpret=_INTERPRET,
    )(im, capT, mask_flat, w1_flat, E, ET)

    scores = scores4.reshape(NJ, B, CJ).transpose(1, 0, 2).reshape(B, B)

    loss2 = pl.pallas_call(
        _loss_kernel,
        out_shape=jax.ShapeDtypeStruct((1, 1), jnp.float32),
        name="hinge_loss",
        interpret=_INTERPRET,
    )(scores)
    return loss2.reshape(())
